# Initial kernel scaffold; baseline (speedup 1.0000x reference)
import jax
import jax.numpy as jnp
from jax.experimental import pallas as pl


def kernel(x, table, W, b):
    raise NotImplementedError("write your pallas kernel here")


# speedup vs baseline: 110.4340x; 110.4340x over previous
import functools

import jax
import jax.numpy as jnp
from jax import lax
from jax.experimental import pallas as pl
from jax.experimental.pallas import tpu as pltpu
from jax.experimental.pallas import tpu_sc as plsc

_VOCAB_PAD = 1024
_NCLS = 7
_CPAD = 8
_NW = 32
_GROUP = 16
_UNROLL = 8


def _prep_body(w_ref, table_ref, b_ref, inv_ref, out_ref):
    m = lax.dot_general(
        w_ref[...], table_ref[...], (((1,), (1,)), ((), ())),
        preferred_element_type=jnp.float32)
    out_ref[...] = (m + b_ref[:, :1]) * inv_ref[0]


def _make_sc_kernel(batch, seq):
    rows_per = batch // _NW
    n_groups = rows_per // _GROUP
    mesh = plsc.VectorSubcoreMesh(core_axis_name="c", subcore_axis_name="s")

    @functools.partial(
        pl.kernel,
        out_type=jax.ShapeDtypeStruct((batch * _NCLS,), jnp.float32),
        mesh=mesh,
        scratch_types=[
            pltpu.VMEM((_CPAD * _VOCAB_PAD,), jnp.float32),
            pltpu.VMEM((rows_per * seq,), jnp.int32),
            pltpu.VMEM((rows_per * _NCLS,), jnp.float32),
        ],
        compiler_params=pltpu.CompilerParams(needs_layout_passes=False),
    )
    def sc_kernel(mt_hbm, x_hbm, out_hbm, mt_v, x_v, out_v):
        wid = lax.axis_index("s") * 2 + lax.axis_index("c")
        base = wid * rows_per
        pltpu.sync_copy(mt_hbm, mt_v)
        pltpu.sync_copy(x_hbm.at[pl.ds(base * seq, rows_per * seq)], x_v)

        lanes = jnp.arange(_GROUP, dtype=jnp.int32)

        def group_body(g, _):
            rows = g * _GROUP + lanes
            rows_flat = rows * seq

            def seq_body(t, accs):
                accs = list(accs)
                for u in range(_UNROLL):
                    idx = plsc.load_gather(
                        x_v, [rows_flat + (t * _UNROLL + u)])
                    for c in range(_NCLS):
                        accs[c] = accs[c] + plsc.load_gather(
                            mt_v, [idx + (c * _VOCAB_PAD)])
                return tuple(accs)

            zero = jnp.zeros((_GROUP,), jnp.float32)
            accs = lax.fori_loop(0, seq // _UNROLL, seq_body,
                                 (zero,) * _NCLS)

            m = accs[0]
            for c in range(1, _NCLS):
                m = jnp.maximum(m, accs[c])
            es = [jnp.exp(a - m) for a in accs]
            tot = es[0]
            for c in range(1, _NCLS):
                tot = tot + es[c]
            out_rows = rows * _NCLS
            for c in range(_NCLS):
                plsc.store_scatter(out_v, [out_rows + c], es[c] / tot)
            return 0

        lax.fori_loop(0, n_groups, group_body, 0)
        pltpu.sync_copy(out_v, out_hbm.at[pl.ds(base * _NCLS,
                                                rows_per * _NCLS)])

    return sc_kernel


def kernel(x, table, W, b):
    batch, seq = x.shape
    x = x.astype(jnp.int32)
    table_p = jnp.pad(table, ((0, _VOCAB_PAD - table.shape[0]), (0, 0)))
    w_p = jnp.pad(W, ((0, _CPAD - W.shape[0]), (0, 0)))
    b_p = jnp.broadcast_to(
        jnp.pad(b, (0, _CPAD - b.shape[0])).reshape(_CPAD, 1), (_CPAD, 128))
    inv = jnp.full((1,), 1.0 / seq, jnp.float32)

    mt = pl.pallas_call(
        _prep_body,
        out_shape=jax.ShapeDtypeStruct((_CPAD, _VOCAB_PAD), jnp.float32),
        in_specs=[
            pl.BlockSpec(memory_space=pltpu.VMEM),
            pl.BlockSpec(memory_space=pltpu.VMEM),
            pl.BlockSpec(memory_space=pltpu.VMEM),
            pl.BlockSpec(memory_space=pltpu.SMEM),
        ],
        out_specs=pl.BlockSpec(memory_space=pltpu.VMEM),
    )(w_p, table_p, b_p, inv)

    out_flat = _make_sc_kernel(batch, seq)(
        mt.reshape(_CPAD * _VOCAB_PAD), x.reshape(batch * seq))
    return out_flat.reshape(batch, _NCLS)


# speedup vs baseline: 112.1821x; 1.0158x over previous
import functools

import jax
import jax.numpy as jnp
from jax import lax
from jax.experimental import pallas as pl
from jax.experimental.pallas import tpu as pltpu
from jax.experimental.pallas import tpu_sc as plsc

_VOCAB_PAD = 1024
_NCLS = 7
_CPAD = 8
_NW = 32
_GROUP = 16
_UNROLL = 8


def _prep_body(w_ref, table_ref, b_ref, inv_ref, out_ref):
    m = lax.dot_general(
        w_ref[...], table_ref[...], (((1,), (1,)), ((), ())),
        preferred_element_type=jnp.float32)
    out_ref[...] = (m + b_ref[:, :1]) * inv_ref[0]


def _make_sc_kernel(batch, seq):
    rows_per = batch // _NW
    n_groups = rows_per // _GROUP
    mesh = plsc.VectorSubcoreMesh(core_axis_name="c", subcore_axis_name="s")

    @functools.partial(
        pl.kernel,
        out_type=jax.ShapeDtypeStruct((batch, _NCLS), jnp.float32),
        mesh=mesh,
        scratch_types=[
            pltpu.VMEM((_CPAD * _VOCAB_PAD,), jnp.float32),
            pltpu.VMEM((rows_per, seq), jnp.int32),
            pltpu.VMEM((rows_per, _NCLS), jnp.float32),
        ],
        compiler_params=pltpu.CompilerParams(
            needs_layout_passes=False, use_tc_tiling_on_sc=False),
    )
    def sc_kernel(mt_hbm, x_hbm, out_hbm, mt_v, x_v, out_v):
        wid = lax.axis_index("s") * 2 + lax.axis_index("c")
        base = wid * rows_per
        pltpu.sync_copy(mt_hbm, mt_v)
        pltpu.sync_copy(x_hbm.at[pl.ds(base, rows_per)], x_v)

        lanes = jnp.arange(_GROUP, dtype=jnp.int32)

        def group_body(g, _):
            rows = g * _GROUP + lanes

            def seq_body(t, accs):
                accs = list(accs)
                for u in range(_UNROLL):
                    col = jnp.zeros((_GROUP,), jnp.int32) + (t * _UNROLL + u)
                    idx = plsc.load_gather(x_v, [rows, col])
                    for c in range(_NCLS):
                        accs[c] = accs[c] + plsc.load_gather(
                            mt_v, [idx + (c * _VOCAB_PAD)])
                return tuple(accs)

            zero = jnp.zeros((_GROUP,), jnp.float32)
            accs = lax.fori_loop(0, seq // _UNROLL, seq_body,
                                 (zero,) * _NCLS)

            m = accs[0]
            for c in range(1, _NCLS):
                m = jnp.maximum(m, accs[c])
            es = [jnp.exp(a - m) for a in accs]
            tot = es[0]
            for c in range(1, _NCLS):
                tot = tot + es[c]
            for c in range(_NCLS):
                cvec = jnp.full((_GROUP,), c, jnp.int32)
                plsc.store_scatter(out_v, [rows, cvec], es[c] / tot)
            return 0

        lax.fori_loop(0, n_groups, group_body, 0)
        pltpu.sync_copy(out_v, out_hbm.at[pl.ds(base, rows_per)])

    return sc_kernel


def kernel(x, table, W, b):
    batch, seq = x.shape
    x = x.astype(jnp.int32)
    table_p = jnp.pad(table, ((0, _VOCAB_PAD - table.shape[0]), (0, 0)))
    w_p = jnp.pad(W, ((0, _CPAD - W.shape[0]), (0, 0)))
    b_p = jnp.broadcast_to(
        jnp.pad(b, (0, _CPAD - b.shape[0])).reshape(_CPAD, 1), (_CPAD, 128))
    inv = jnp.full((1,), 1.0 / seq, jnp.float32)

    mt = pl.pallas_call(
        _prep_body,
        out_shape=jax.ShapeDtypeStruct((_CPAD, _VOCAB_PAD), jnp.float32),
        in_specs=[
            pl.BlockSpec(memory_space=pltpu.VMEM),
            pl.BlockSpec(memory_space=pltpu.VMEM),
            pl.BlockSpec(memory_space=pltpu.VMEM),
            pl.BlockSpec(memory_space=pltpu.SMEM),
        ],
        out_specs=pl.BlockSpec(memory_space=pltpu.VMEM),
    )(w_p, table_p, b_p, inv)

    return _make_sc_kernel(batch, seq)(
        mt.reshape(_CPAD * _VOCAB_PAD), x)
